# Initial kernel scaffold; baseline (speedup 1.0000x reference)
import jax
import jax.numpy as jnp
from jax.experimental import pallas as pl


def kernel(logits, labels):
    raise NotImplementedError("write your pallas kernel here")


# speedup vs baseline: 1.1217x; 1.1217x over previous
import functools

import jax
import jax.numpy as jnp
from jax.experimental import pallas as pl

_S = 64.0
_M = 0.35

_ROWS = 1024
_BLOCK_COLS = 2048


def _scale_body(lab_ref, x_ref, o_ref):
    j = pl.program_id(0)
    x = x_ref[...]
    lab = lab_ref[...]
    col = jax.lax.broadcasted_iota(jnp.int32, x.shape, 1) + j * _BLOCK_COLS
    delta = jnp.where(col == lab, -_M * _S, 0.0).astype(x.dtype)
    o_ref[...] = x * _S + delta


@jax.jit
def kernel(logits, labels):
    rows, cols = logits.shape
    lab2d = labels.astype(jnp.int32).reshape(rows, 1)
    grid = pl.cdiv(cols, _BLOCK_COLS)
    return pl.pallas_call(
        _scale_body,
        grid=(grid,),
        in_specs=[
            pl.BlockSpec((rows, 1), lambda j: (0, 0)),
            pl.BlockSpec((rows, _BLOCK_COLS), lambda j: (0, j)),
        ],
        out_specs=pl.BlockSpec((rows, _BLOCK_COLS), lambda j: (0, j)),
        out_shape=jax.ShapeDtypeStruct((rows, cols), logits.dtype),
    )(lab2d, logits)
